# baseline (speedup 1.0000x reference)
import functools

import jax
import jax.numpy as jnp
from jax.experimental import pallas as pl
from jax.experimental.pallas import tpu as pltpu

N = 10000
E = 160000
D = 128
L = 2
COORD_RANGE = 12.0 / L

EB = 2000


def _silu(v):
    return v * jax.nn.sigmoid(v)


def _edge_mlp_body(z_ref, rad_ref, dorg_ref, wd_ref, wo_ref, w2_ref, b2_ref,
                   w3_ref, t_ref):
    rad = rad_ref[...]
    z = z_ref[...] + rad * wd_ref[...] + dorg_ref[...] * wo_ref[...]
    u = _silu(z)
    v = _silu(jnp.dot(u, w2_ref[...], preferred_element_type=jnp.float32)
              + b2_ref[...])
    s = jnp.dot(v, w3_ref[...], preferred_element_type=jnp.float32)
    t_ref[...] = (COORD_RANGE * jnp.tanh(s)
                  / (jnp.sqrt(rad + 1e-8) + 1.0))


@jax.jit
def _edge_mlp(z, rad, dorg, wd, wo, w2, b2, w3):
    grid = (E // EB,)
    return pl.pallas_call(
        _edge_mlp_body,
        grid=grid,
        in_specs=[
            pl.BlockSpec((EB, D), lambda i: (i, 0)),
            pl.BlockSpec((EB, 1), lambda i: (i, 0)),
            pl.BlockSpec((EB, 1), lambda i: (i, 0)),
            pl.BlockSpec((1, D), lambda i: (0, 0)),
            pl.BlockSpec((1, D), lambda i: (0, 0)),
            pl.BlockSpec((D, D), lambda i: (0, 0)),
            pl.BlockSpec((1, D), lambda i: (0, 0)),
            pl.BlockSpec((D, 1), lambda i: (0, 0)),
        ],
        out_specs=pl.BlockSpec((EB, 1), lambda i: (i, 0)),
        out_shape=jax.ShapeDtypeStruct((E, 1), jnp.float32),
    )(z, rad, dorg, wd, wo, w2, b2, w3)


def kernel(h, x, distance_org, edge_index, edg1_w, edg1_b, edg2_w, edg2_b,
           edgi_w, edgi_b, node1_w, node1_b, node2_w, node2_b, cor1_w,
           cor1_b, cor2_w, cor2_b, cor3_w):
    row = edge_index[0]
    col = edge_index[1]
    for l in range(L):
        coord_diff = x[row] - x[col]
        radial = jnp.sum(coord_diff ** 2, axis=1, keepdims=True)
        agg = jax.ops.segment_sum(radial, row, num_segments=N) / 100.0
        hn = _silu(h @ node1_w[l][:D] + agg * node1_w[l][D:D + 1]
                   + node1_b[l])
        hn = hn @ node2_w[l] + node2_b[l]
        h = h + hn
        w1 = cor1_w[l]
        a = h @ w1[:D] + cor1_b[l]
        b = h @ w1[D:2 * D]
        z = a[row] + b[col]
        t = _edge_mlp(z, radial, distance_org,
                      w1[2 * D:2 * D + 1], w1[2 * D + 1:2 * D + 2],
                      cor2_w[l], cor2_b[l].reshape(1, D), cor3_w[l])
        trans = coord_diff * t
        x = x + jax.ops.segment_sum(trans, row, num_segments=N) / 100.0
    return (h, x)


# speedup vs baseline: 2.9279x; 2.9279x over previous
import functools

import jax
import jax.numpy as jnp
from jax import lax
from jax.experimental import pallas as pl
from jax.experimental.pallas import tpu as pltpu
from jax.experimental.pallas import tpu_sc as plsc

N = 10000
D = 128
L = 2
COORD_RANGE = 12.0 / L

NC = 2
NS = 16
NW = NC * NS

E = 160000
EP = 163840
EW = EP // NW
NP = 10240
SINK = N

EB = 2048
NB = 2048
SC_CHUNK = 512

_MESH = plsc.VectorSubcoreMesh(core_axis_name="c", subcore_axis_name="s")
_SC_PARAMS = pltpu.CompilerParams(needs_layout_passes=False)


def _wid():
    return lax.axis_index("s") * NC + lax.axis_index("c")


def _silu(v):
    return v * jax.nn.sigmoid(v)


def _s1_body(xt_hbm, row_hbm, col_hbm, rad_hbm, cd0_hbm, cd1_hbm, cd2_hbm,
             aggp_hbm, xt_v, row_v, col_v, rad_v, c0_v, c1_v, c2_v, agg_v,
             sem):
    wid = _wid()
    base = wid * EW
    pltpu.sync_copy(xt_hbm, xt_v)
    pltpu.sync_copy(row_hbm.at[pl.ds(base, EW)], row_v)
    pltpu.sync_copy(col_hbm.at[pl.ds(base, EW)], col_v)

    def zero(i, c):
        agg_v[pl.ds(i * 16, 16)] = jnp.zeros((16,), jnp.float32)
        return c
    lax.fori_loop(0, NP // 16, zero, 0)

    cd_refs = (c0_v, c1_v, c2_v)

    def body(g, c):
        sl = pl.ds(g * 16, 16)
        r = row_v[sl]
        cc = col_v[sl]
        rad = jnp.zeros((16,), jnp.float32)
        for j in range(3):
            off = jnp.int32(j * NP)
            dj = (plsc.load_gather(xt_v, [r + off])
                  - plsc.load_gather(xt_v, [cc + off]))
            cd_refs[j][sl] = dj
            rad = rad + dj * dj
        rad_v[sl] = rad
        plsc.addupdate_scatter(agg_v, [r], rad)
        return c
    lax.fori_loop(0, EW // 16, body, 0)

    pltpu.sync_copy(rad_v, rad_hbm.at[pl.ds(base, EW)])
    pltpu.sync_copy(c0_v, cd0_hbm.at[pl.ds(base, EW)])
    pltpu.sync_copy(c1_v, cd1_hbm.at[pl.ds(base, EW)])
    pltpu.sync_copy(c2_v, cd2_hbm.at[pl.ds(base, EW)])
    pltpu.sync_copy(agg_v, aggp_hbm.at[pl.ds(wid * NP, NP)])


@jax.jit
def _s1(xt, row, col):
    f = pl.kernel(
        _s1_body,
        out_type=[
            jax.ShapeDtypeStruct((EP,), jnp.float32),
            jax.ShapeDtypeStruct((EP,), jnp.float32),
            jax.ShapeDtypeStruct((EP,), jnp.float32),
            jax.ShapeDtypeStruct((EP,), jnp.float32),
            jax.ShapeDtypeStruct((NW * NP,), jnp.float32),
        ],
        mesh=_MESH,
        compiler_params=_SC_PARAMS,
        scratch_types=[
            pltpu.VMEM((3 * NP,), jnp.float32),
            pltpu.VMEM((EW,), jnp.int32),
            pltpu.VMEM((EW,), jnp.int32),
            pltpu.VMEM((EW,), jnp.float32),
            pltpu.VMEM((EW,), jnp.float32),
            pltpu.VMEM((EW,), jnp.float32),
            pltpu.VMEM((EW,), jnp.float32),
            pltpu.VMEM((NP,), jnp.float32),
            pltpu.SemaphoreType.DMA,
        ],
    )
    return f(xt, row, col)


def _s2_body(a_hbm, b_hbm, row_hbm, col_hbm, z_hbm, idx_v, z_v, sem):
    base = _wid() * EW

    def chunk(k, c):
        off = base + k * SC_CHUNK
        pltpu.sync_copy(row_hbm.at[pl.ds(off, SC_CHUNK)], idx_v)
        pltpu.async_copy(a_hbm.at[idx_v], z_v, sem).wait()
        pltpu.sync_copy(col_hbm.at[pl.ds(off, SC_CHUNK)], idx_v)
        pltpu.async_copy(b_hbm.at[idx_v], z_v, sem, add=True).wait()
        pltpu.sync_copy(z_v, z_hbm.at[pl.ds(off, SC_CHUNK)])
        return c
    lax.fori_loop(0, EW // SC_CHUNK, chunk, 0)


@jax.jit
def _s2(a, b, row, col):
    f = pl.kernel(
        _s2_body,
        out_type=jax.ShapeDtypeStruct((EP, D), jnp.float32),
        mesh=_MESH,
        compiler_params=_SC_PARAMS,
        scratch_types=[
            pltpu.VMEM((SC_CHUNK,), jnp.int32),
            pltpu.VMEM((SC_CHUNK, D), jnp.float32),
            pltpu.SemaphoreType.DMA,
        ],
    )
    return f(a, b, row, col)


def _s3_body(t_hbm, cd0_hbm, cd1_hbm, cd2_hbm, row_hbm, tp_hbm,
             t_v, c0_v, c1_v, c2_v, row_v, a0_v, a1_v, a2_v, sem):
    wid = _wid()
    base = wid * EW
    pltpu.sync_copy(t_hbm.at[pl.ds(base, EW)], t_v)
    pltpu.sync_copy(cd0_hbm.at[pl.ds(base, EW)], c0_v)
    pltpu.sync_copy(cd1_hbm.at[pl.ds(base, EW)], c1_v)
    pltpu.sync_copy(cd2_hbm.at[pl.ds(base, EW)], c2_v)
    pltpu.sync_copy(row_hbm.at[pl.ds(base, EW)], row_v)

    acc_refs = (a0_v, a1_v, a2_v)
    cd_refs = (c0_v, c1_v, c2_v)

    def zero(i, c):
        for j in range(3):
            acc_refs[j][pl.ds(i * 16, 16)] = jnp.zeros((16,), jnp.float32)
        return c
    lax.fori_loop(0, NP // 16, zero, 0)

    def body(g, c):
        sl = pl.ds(g * 16, 16)
        r = row_v[sl]
        tv = t_v[sl]
        for j in range(3):
            plsc.addupdate_scatter(acc_refs[j], [r], cd_refs[j][sl] * tv)
        return c
    lax.fori_loop(0, EW // 16, body, 0)

    for j in range(3):
        pltpu.sync_copy(acc_refs[j],
                        tp_hbm.at[pl.ds((j * NW + wid) * NP, NP)])


@jax.jit
def _s3(t, cd0, cd1, cd2, row):
    f = pl.kernel(
        _s3_body,
        out_type=jax.ShapeDtypeStruct((3 * NW * NP,), jnp.float32),
        mesh=_MESH,
        compiler_params=_SC_PARAMS,
        scratch_types=[
            pltpu.VMEM((EW,), jnp.float32),
            pltpu.VMEM((EW,), jnp.float32),
            pltpu.VMEM((EW,), jnp.float32),
            pltpu.VMEM((EW,), jnp.float32),
            pltpu.VMEM((EW,), jnp.int32),
            pltpu.VMEM((NP,), jnp.float32),
            pltpu.VMEM((NP,), jnp.float32),
            pltpu.VMEM((NP,), jnp.float32),
            pltpu.SemaphoreType.DMA,
        ],
    )
    return f(t, cd0, cd1, cd2, row)


def _t1_body(h_ref, aggt_ref, n1h_ref, n1a_ref, n1b_ref, n2w_ref, n2b_ref,
             c1a_ref, c1b_ref, c1bias_ref, hn_ref, a_ref, b_ref):
    hv = h_ref[...]
    agg = jnp.sum(aggt_ref[...], axis=1, keepdims=True) * 0.01
    z = (jnp.dot(hv, n1h_ref[...], preferred_element_type=jnp.float32)
         + agg * n1a_ref[...] + n1b_ref[...])
    u = _silu(z)
    hn = jnp.dot(u, n2w_ref[...], preferred_element_type=jnp.float32) \
        + n2b_ref[...]
    hnew = hv + hn
    hn_ref[...] = hnew
    a_ref[...] = (jnp.dot(hnew, c1a_ref[...],
                          preferred_element_type=jnp.float32)
                  + c1bias_ref[...])
    b_ref[...] = jnp.dot(hnew, c1b_ref[...],
                         preferred_element_type=jnp.float32)


@jax.jit
def _t1(h, aggt, n1h, n1a, n1b, n2w, n2b, c1a, c1b, c1bias):
    grid = (NP // NB,)
    return pl.pallas_call(
        _t1_body,
        grid=grid,
        in_specs=[
            pl.BlockSpec((NB, D), lambda i: (i, 0)),
            pl.BlockSpec((NB, NW), lambda i: (i, 0)),
            pl.BlockSpec((D, D), lambda i: (0, 0)),
            pl.BlockSpec((1, D), lambda i: (0, 0)),
            pl.BlockSpec((1, D), lambda i: (0, 0)),
            pl.BlockSpec((D, D), lambda i: (0, 0)),
            pl.BlockSpec((1, D), lambda i: (0, 0)),
            pl.BlockSpec((D, D), lambda i: (0, 0)),
            pl.BlockSpec((D, D), lambda i: (0, 0)),
            pl.BlockSpec((1, D), lambda i: (0, 0)),
        ],
        out_specs=[
            pl.BlockSpec((NB, D), lambda i: (i, 0)),
            pl.BlockSpec((NB, D), lambda i: (i, 0)),
            pl.BlockSpec((NB, D), lambda i: (i, 0)),
        ],
        out_shape=[
            jax.ShapeDtypeStruct((NP, D), jnp.float32),
            jax.ShapeDtypeStruct((NP, D), jnp.float32),
            jax.ShapeDtypeStruct((NP, D), jnp.float32),
        ],
    )(h, aggt, n1h, n1a, n1b, n2w, n2b, c1a, c1b, c1bias)


def _t2_body(z_ref, rad_ref, dorg_ref, wd_ref, wo_ref, w2_ref, b2_ref,
             w3_ref, t_ref):
    rad = rad_ref[...]
    z = z_ref[...] + rad * wd_ref[...] + dorg_ref[...] * wo_ref[...]
    u = _silu(z)
    v = _silu(jnp.dot(u, w2_ref[...], preferred_element_type=jnp.float32)
              + b2_ref[...])
    s = jnp.dot(v, w3_ref[...], preferred_element_type=jnp.float32)
    t_ref[...] = (COORD_RANGE * jnp.tanh(s)
                  / (jnp.sqrt(rad + 1e-8) + 1.0))


@jax.jit
def _t2(z, rad, dorg, wd, wo, w2, b2, w3):
    grid = (EP // EB,)
    return pl.pallas_call(
        _t2_body,
        grid=grid,
        in_specs=[
            pl.BlockSpec((EB, D), lambda i: (i, 0)),
            pl.BlockSpec((EB, 1), lambda i: (i, 0)),
            pl.BlockSpec((EB, 1), lambda i: (i, 0)),
            pl.BlockSpec((1, D), lambda i: (0, 0)),
            pl.BlockSpec((1, D), lambda i: (0, 0)),
            pl.BlockSpec((D, D), lambda i: (0, 0)),
            pl.BlockSpec((1, D), lambda i: (0, 0)),
            pl.BlockSpec((D, 1), lambda i: (0, 0)),
        ],
        out_specs=pl.BlockSpec((EB, 1), lambda i: (i, 0)),
        out_shape=jax.ShapeDtypeStruct((EP, 1), jnp.float32),
    )(z, rad, dorg, wd, wo, w2, b2, w3)


def _tx_body(xt_ref, tp_ref, xo_ref):
    s = jnp.sum(tp_ref[...], axis=1)
    xo_ref[...] = xt_ref[...] + s * 0.01


@jax.jit
def _tx(xt, tp):
    grid = (NP // NB,)
    return pl.pallas_call(
        _tx_body,
        grid=grid,
        in_specs=[
            pl.BlockSpec((3, NB), lambda i: (0, i)),
            pl.BlockSpec((3, NW, NB), lambda i: (0, 0, i)),
        ],
        out_specs=pl.BlockSpec((3, NB), lambda i: (0, i)),
        out_shape=jax.ShapeDtypeStruct((3, NP), jnp.float32),
    )(xt, tp)


@jax.jit
def _impl(h, x, distance_org, edge_index, node1_w, node1_b, node2_w,
          node2_b, cor1_w, cor1_b, cor2_w, cor2_b, cor3_w):
    row = jnp.pad(edge_index[0], (0, EP - E), constant_values=SINK)
    col = jnp.pad(edge_index[1], (0, EP - E), constant_values=SINK)
    dorg = jnp.pad(distance_org, ((0, EP - E), (0, 0)))
    xt = jnp.pad(x.T, ((0, 0), (0, NP - N)))
    hp = jnp.pad(h, ((0, NP - N), (0, 0)))
    for l in range(L):
        rad, cd0, cd1, cd2, aggp = _s1(xt.reshape(3 * NP), row, col)
        aggt = jnp.transpose(aggp.reshape(NW, NP))
        w1 = cor1_w[l]
        hp, a, b = _t1(hp, aggt, node1_w[l][:D], node1_w[l][D:D + 1],
                       node1_b[l].reshape(1, D), node2_w[l],
                       node2_b[l].reshape(1, D), w1[:D], w1[D:2 * D],
                       cor1_b[l].reshape(1, D))
        z = _s2(a, b, row, col)
        t = _t2(z, rad.reshape(EP, 1), dorg, w1[2 * D:2 * D + 1],
                w1[2 * D + 1:2 * D + 2], cor2_w[l],
                cor2_b[l].reshape(1, D), cor3_w[l])
        tp = _s3(t.reshape(EP), cd0, cd1, cd2, row)
        xt = _tx(xt, tp.reshape(3, NW, NP))
    return hp[:N], xt[:, :N].T


def kernel(h, x, distance_org, edge_index, edg1_w, edg1_b, edg2_w, edg2_b,
           edgi_w, edgi_b, node1_w, node1_b, node2_w, node2_b, cor1_w,
           cor1_b, cor2_w, cor2_b, cor3_w):
    return _impl(h, x, distance_org, edge_index, node1_w, node1_b,
                 node2_w, node2_b, cor1_w, cor1_b, cor2_w, cor2_b, cor3_w)


# speedup vs baseline: 3.8448x; 1.3132x over previous
import functools

import jax
import jax.numpy as jnp
from jax import lax
from jax.experimental import pallas as pl
from jax.experimental.pallas import tpu as pltpu
from jax.experimental.pallas import tpu_sc as plsc

N = 10000
D = 128
L = 2
COORD_RANGE = 12.0 / L

NC = 2
NS = 16
NW = NC * NS

E = 160000
EP = 163840
EW = EP // NW
NP = 10240
SINK = N

EB = 2048
NB = 2048
SC_CHUNK = 256

_MESH = plsc.VectorSubcoreMesh(core_axis_name="c", subcore_axis_name="s")
_SC_PARAMS = pltpu.CompilerParams(needs_layout_passes=False)


def _wid():
    return lax.axis_index("s") * NC + lax.axis_index("c")


def _silu(v):
    return v * jax.nn.sigmoid(v)


def _s1_body(xt_hbm, row_hbm, col_hbm, rad_hbm, cd0_hbm, cd1_hbm, cd2_hbm,
             aggp_hbm, xt_v, row_v, col_v, rad_v, c0_v, c1_v, c2_v, agg_v,
             sem):
    wid = _wid()
    base = wid * EW
    pltpu.sync_copy(xt_hbm, xt_v)
    pltpu.sync_copy(row_hbm.at[pl.ds(base, EW)], row_v)
    pltpu.sync_copy(col_hbm.at[pl.ds(base, EW)], col_v)

    def zero(i, c):
        agg_v[pl.ds(i * 16, 16)] = jnp.zeros((16,), jnp.float32)
        return c
    lax.fori_loop(0, NP // 16, zero, 0)

    cd_refs = (c0_v, c1_v, c2_v)

    def body(g, c):
        sl = pl.ds(g * 16, 16)
        r = row_v[sl]
        cc = col_v[sl]
        rad = jnp.zeros((16,), jnp.float32)
        for j in range(3):
            off = jnp.int32(j * NP)
            dj = (plsc.load_gather(xt_v, [r + off])
                  - plsc.load_gather(xt_v, [cc + off]))
            cd_refs[j][sl] = dj
            rad = rad + dj * dj
        rad_v[sl] = rad
        plsc.addupdate_scatter(agg_v, [r], rad)
        return c
    lax.fori_loop(0, EW // 16, body, 0)

    pltpu.sync_copy(rad_v, rad_hbm.at[pl.ds(base, EW)])
    pltpu.sync_copy(c0_v, cd0_hbm.at[pl.ds(base, EW)])
    pltpu.sync_copy(c1_v, cd1_hbm.at[pl.ds(base, EW)])
    pltpu.sync_copy(c2_v, cd2_hbm.at[pl.ds(base, EW)])
    pltpu.sync_copy(agg_v, aggp_hbm.at[pl.ds(wid * NP, NP)])


@jax.jit
def _s1(xt, row, col):
    f = pl.kernel(
        _s1_body,
        out_type=[
            jax.ShapeDtypeStruct((EP,), jnp.float32),
            jax.ShapeDtypeStruct((EP,), jnp.float32),
            jax.ShapeDtypeStruct((EP,), jnp.float32),
            jax.ShapeDtypeStruct((EP,), jnp.float32),
            jax.ShapeDtypeStruct((NW * NP,), jnp.float32),
        ],
        mesh=_MESH,
        compiler_params=_SC_PARAMS,
        scratch_types=[
            pltpu.VMEM((3 * NP,), jnp.float32),
            pltpu.VMEM((EW,), jnp.int32),
            pltpu.VMEM((EW,), jnp.int32),
            pltpu.VMEM((EW,), jnp.float32),
            pltpu.VMEM((EW,), jnp.float32),
            pltpu.VMEM((EW,), jnp.float32),
            pltpu.VMEM((EW,), jnp.float32),
            pltpu.VMEM((NP,), jnp.float32),
            pltpu.SemaphoreType.DMA,
        ],
    )
    return f(xt, row, col)


NCH = EW // SC_CHUNK


def _s2_body(a_hbm, b_hbm, row_hbm, col_hbm, z_hbm, row_v, col_v,
             z0, z1, z2, sa0, sa1, sa2, sb0, sb1, sb2, so0, so1, so2):
    base = _wid() * EW
    pltpu.sync_copy(row_hbm.at[pl.ds(base, EW)], row_v)
    pltpu.sync_copy(col_hbm.at[pl.ds(base, EW)], col_v)
    zb = (z0, z1, z2)
    sa = (sa0, sa1, sa2)
    sb = (sb0, sb1, sb2)
    so = (so0, so1, so2)
    da, db, do_ = {}, {}, {}

    def start_a(k):
        j = k % 3
        idx = row_v.at[pl.ds(k * SC_CHUNK, SC_CHUNK)]
        da[k] = pltpu.async_copy(a_hbm.at[idx], zb[j], sa[j])

    start_a(0)
    for k in range(NCH):
        j = k % 3
        if k + 1 < NCH:
            if k >= 2:
                do_[k - 2].wait()
            start_a(k + 1)
        da[k].wait()
        idx = col_v.at[pl.ds(k * SC_CHUNK, SC_CHUNK)]
        db[k] = pltpu.async_copy(b_hbm.at[idx], zb[j], sb[j], add=True)
        db[k].wait()
        do_[k] = pltpu.async_copy(
            zb[j], z_hbm.at[pl.ds(base + k * SC_CHUNK, SC_CHUNK)], so[j])
    for k in range(max(0, NCH - 3), NCH):
        do_[k].wait()


@jax.jit
def _s2(a, b, row, col):
    f = pl.kernel(
        _s2_body,
        out_type=jax.ShapeDtypeStruct((EP, D), jnp.float32),
        mesh=_MESH,
        compiler_params=_SC_PARAMS,
        scratch_types=[
            pltpu.VMEM((EW,), jnp.int32),
            pltpu.VMEM((EW,), jnp.int32),
            pltpu.VMEM((SC_CHUNK, D), jnp.float32),
            pltpu.VMEM((SC_CHUNK, D), jnp.float32),
            pltpu.VMEM((SC_CHUNK, D), jnp.float32),
            pltpu.SemaphoreType.DMA,
            pltpu.SemaphoreType.DMA,
            pltpu.SemaphoreType.DMA,
            pltpu.SemaphoreType.DMA,
            pltpu.SemaphoreType.DMA,
            pltpu.SemaphoreType.DMA,
            pltpu.SemaphoreType.DMA,
            pltpu.SemaphoreType.DMA,
            pltpu.SemaphoreType.DMA,
        ],
    )
    return f(a, b, row, col)


def _s3_body(t_hbm, cd0_hbm, cd1_hbm, cd2_hbm, row_hbm, tp_hbm,
             t_v, c0_v, c1_v, c2_v, row_v, a0_v, a1_v, a2_v, sem):
    wid = _wid()
    base = wid * EW
    pltpu.sync_copy(t_hbm.at[pl.ds(base, EW)], t_v)
    pltpu.sync_copy(cd0_hbm.at[pl.ds(base, EW)], c0_v)
    pltpu.sync_copy(cd1_hbm.at[pl.ds(base, EW)], c1_v)
    pltpu.sync_copy(cd2_hbm.at[pl.ds(base, EW)], c2_v)
    pltpu.sync_copy(row_hbm.at[pl.ds(base, EW)], row_v)

    acc_refs = (a0_v, a1_v, a2_v)
    cd_refs = (c0_v, c1_v, c2_v)

    def zero(i, c):
        for j in range(3):
            acc_refs[j][pl.ds(i * 16, 16)] = jnp.zeros((16,), jnp.float32)
        return c
    lax.fori_loop(0, NP // 16, zero, 0)

    def body(g, c):
        sl = pl.ds(g * 16, 16)
        r = row_v[sl]
        tv = t_v[sl]
        for j in range(3):
            plsc.addupdate_scatter(acc_refs[j], [r], cd_refs[j][sl] * tv)
        return c
    lax.fori_loop(0, EW // 16, body, 0)

    for j in range(3):
        pltpu.sync_copy(acc_refs[j],
                        tp_hbm.at[pl.ds((j * NW + wid) * NP, NP)])


@jax.jit
def _s3(t, cd0, cd1, cd2, row):
    f = pl.kernel(
        _s3_body,
        out_type=jax.ShapeDtypeStruct((3 * NW * NP,), jnp.float32),
        mesh=_MESH,
        compiler_params=_SC_PARAMS,
        scratch_types=[
            pltpu.VMEM((EW,), jnp.float32),
            pltpu.VMEM((EW,), jnp.float32),
            pltpu.VMEM((EW,), jnp.float32),
            pltpu.VMEM((EW,), jnp.float32),
            pltpu.VMEM((EW,), jnp.int32),
            pltpu.VMEM((NP,), jnp.float32),
            pltpu.VMEM((NP,), jnp.float32),
            pltpu.VMEM((NP,), jnp.float32),
            pltpu.SemaphoreType.DMA,
        ],
    )
    return f(t, cd0, cd1, cd2, row)


def _t1_body(h_ref, aggt_ref, n1h_ref, n1a_ref, n1b_ref, n2w_ref, n2b_ref,
             c1a_ref, c1b_ref, c1bias_ref, hn_ref, a_ref, b_ref):
    hv = h_ref[...]
    agg = jnp.sum(aggt_ref[...], axis=1, keepdims=True) * 0.01
    z = (jnp.dot(hv, n1h_ref[...], preferred_element_type=jnp.float32)
         + agg * n1a_ref[...] + n1b_ref[...])
    u = _silu(z)
    hn = jnp.dot(u, n2w_ref[...], preferred_element_type=jnp.float32) \
        + n2b_ref[...]
    hnew = hv + hn
    hn_ref[...] = hnew
    a_ref[...] = (jnp.dot(hnew, c1a_ref[...],
                          preferred_element_type=jnp.float32)
                  + c1bias_ref[...])
    b_ref[...] = jnp.dot(hnew, c1b_ref[...],
                         preferred_element_type=jnp.float32)


@jax.jit
def _t1(h, aggt, n1h, n1a, n1b, n2w, n2b, c1a, c1b, c1bias):
    grid = (NP // NB,)
    return pl.pallas_call(
        _t1_body,
        grid=grid,
        in_specs=[
            pl.BlockSpec((NB, D), lambda i: (i, 0)),
            pl.BlockSpec((NB, NW), lambda i: (i, 0)),
            pl.BlockSpec((D, D), lambda i: (0, 0)),
            pl.BlockSpec((1, D), lambda i: (0, 0)),
            pl.BlockSpec((1, D), lambda i: (0, 0)),
            pl.BlockSpec((D, D), lambda i: (0, 0)),
            pl.BlockSpec((1, D), lambda i: (0, 0)),
            pl.BlockSpec((D, D), lambda i: (0, 0)),
            pl.BlockSpec((D, D), lambda i: (0, 0)),
            pl.BlockSpec((1, D), lambda i: (0, 0)),
        ],
        out_specs=[
            pl.BlockSpec((NB, D), lambda i: (i, 0)),
            pl.BlockSpec((NB, D), lambda i: (i, 0)),
            pl.BlockSpec((NB, D), lambda i: (i, 0)),
        ],
        out_shape=[
            jax.ShapeDtypeStruct((NP, D), jnp.float32),
            jax.ShapeDtypeStruct((NP, D), jnp.float32),
            jax.ShapeDtypeStruct((NP, D), jnp.float32),
        ],
    )(h, aggt, n1h, n1a, n1b, n2w, n2b, c1a, c1b, c1bias)


def _t2_body(z_ref, rad_ref, dorg_ref, wd_ref, wo_ref, w2_ref, b2_ref,
             w3_ref, t_ref):
    rad = rad_ref[...]
    z = z_ref[...] + rad * wd_ref[...] + dorg_ref[...] * wo_ref[...]
    u = _silu(z)
    v = _silu(jnp.dot(u, w2_ref[...], preferred_element_type=jnp.float32)
              + b2_ref[...])
    s = jnp.dot(v, w3_ref[...], preferred_element_type=jnp.float32)
    t_ref[...] = (COORD_RANGE * jnp.tanh(s)
                  / (jnp.sqrt(rad + 1e-8) + 1.0))


@jax.jit
def _t2(z, rad, dorg, wd, wo, w2, b2, w3):
    grid = (EP // EB,)
    return pl.pallas_call(
        _t2_body,
        grid=grid,
        in_specs=[
            pl.BlockSpec((EB, D), lambda i: (i, 0)),
            pl.BlockSpec((EB, 1), lambda i: (i, 0)),
            pl.BlockSpec((EB, 1), lambda i: (i, 0)),
            pl.BlockSpec((1, D), lambda i: (0, 0)),
            pl.BlockSpec((1, D), lambda i: (0, 0)),
            pl.BlockSpec((D, D), lambda i: (0, 0)),
            pl.BlockSpec((1, D), lambda i: (0, 0)),
            pl.BlockSpec((D, 1), lambda i: (0, 0)),
        ],
        out_specs=pl.BlockSpec((EB, 1), lambda i: (i, 0)),
        out_shape=jax.ShapeDtypeStruct((EP, 1), jnp.float32),
    )(z, rad, dorg, wd, wo, w2, b2, w3)


def _tx_body(xt_ref, tp_ref, xo_ref):
    s = jnp.sum(tp_ref[...], axis=1)
    xo_ref[...] = xt_ref[...] + s * 0.01


@jax.jit
def _tx(xt, tp):
    grid = (NP // NB,)
    return pl.pallas_call(
        _tx_body,
        grid=grid,
        in_specs=[
            pl.BlockSpec((3, NB), lambda i: (0, i)),
            pl.BlockSpec((3, NW, NB), lambda i: (0, 0, i)),
        ],
        out_specs=pl.BlockSpec((3, NB), lambda i: (0, i)),
        out_shape=jax.ShapeDtypeStruct((3, NP), jnp.float32),
    )(xt, tp)


@jax.jit
def _impl(h, x, distance_org, edge_index, node1_w, node1_b, node2_w,
          node2_b, cor1_w, cor1_b, cor2_w, cor2_b, cor3_w):
    row = jnp.pad(edge_index[0], (0, EP - E), constant_values=SINK)
    col = jnp.pad(edge_index[1], (0, EP - E), constant_values=SINK)
    dorg = jnp.pad(distance_org, ((0, EP - E), (0, 0)))
    xt = jnp.pad(x.T, ((0, 0), (0, NP - N)))
    hp = jnp.pad(h, ((0, NP - N), (0, 0)))
    for l in range(L):
        rad, cd0, cd1, cd2, aggp = _s1(xt.reshape(3 * NP), row, col)
        aggt = jnp.transpose(aggp.reshape(NW, NP))
        w1 = cor1_w[l]
        hp, a, b = _t1(hp, aggt, node1_w[l][:D], node1_w[l][D:D + 1],
                       node1_b[l].reshape(1, D), node2_w[l],
                       node2_b[l].reshape(1, D), w1[:D], w1[D:2 * D],
                       cor1_b[l].reshape(1, D))
        z = _s2(a, b, row, col)
        t = _t2(z, rad.reshape(EP, 1), dorg, w1[2 * D:2 * D + 1],
                w1[2 * D + 1:2 * D + 2], cor2_w[l],
                cor2_b[l].reshape(1, D), cor3_w[l])
        tp = _s3(t.reshape(EP), cd0, cd1, cd2, row)
        xt = _tx(xt, tp.reshape(3, NW, NP))
    return hp[:N], xt[:, :N].T


def kernel(h, x, distance_org, edge_index, edg1_w, edg1_b, edg2_w, edg2_b,
           edgi_w, edgi_b, node1_w, node1_b, node2_w, node2_b, cor1_w,
           cor1_b, cor2_w, cor2_b, cor3_w):
    return _impl(h, x, distance_org, edge_index, node1_w, node1_b,
                 node2_w, node2_b, cor1_w, cor1_b, cor2_w, cor2_b, cor3_w)
